# Initial kernel scaffold; baseline (speedup 1.0000x reference)
import jax
import jax.numpy as jnp
from jax.experimental import pallas as pl


def kernel(x):
    raise NotImplementedError("write your pallas kernel here")


# speedup vs baseline: 17.8695x; 17.8695x over previous
import functools

import jax
import jax.numpy as jnp
from jax import lax
from jax.experimental import pallas as pl
from jax.experimental.pallas import tpu as pltpu
from jax.experimental.pallas import tpu_sc as plsc

_B, _C = 2, 96
_OH = _OW = 27
_KH = _KW = 16
_H = _W = 224
_N_IMG = _B * _C
_ROW_ELEMS = _OW * _KH * _KW
_IMG_OUT = _H * _W
_N_WORKERS = 32
_IMGS_PER_WORKER = _N_IMG // _N_WORKERS


def _fold_sc(xr):
    mesh = plsc.VectorSubcoreMesh(core_axis_name="c", subcore_axis_name="s")

    @functools.partial(
        pl.kernel,
        out_type=jax.ShapeDtypeStruct((_N_IMG, _IMG_OUT), jnp.float32),
        mesh=mesh,
        scratch_types=[
            pltpu.VMEM((_ROW_ELEMS,), jnp.float32),
            pltpu.VMEM((_IMG_OUT,), jnp.float32),
        ],
    )
    def k(x_hbm, out_hbm, xrow, obuf):
        wid = lax.axis_index("s") * 2 + lax.axis_index("c")
        zeros16 = jnp.zeros((16,), jnp.float32)

        def zero_body(t, carry):
            obuf[pl.ds(t * 16, 16)] = zeros16
            return carry

        def run_image(img):
            lax.fori_loop(0, _IMG_OUT // 16, zero_body, 0)

            def row_body(i, carry):
                pltpu.sync_copy(x_hbm.at[img, i], xrow)

                def kh_body(kh, inner):
                    base_dst = (8 * i + kh) * _W
                    base_src = kh * _KW
                    for j in range(_OW):
                        v = xrow[pl.ds(base_src + j * (_KH * _KW), 16)]
                        plsc.addupdate(obuf.at[pl.ds(base_dst + 8 * j, 16)], v)
                    return inner

                lax.fori_loop(0, _KH, kh_body, 0)
                return carry

            lax.fori_loop(0, _OH, row_body, 0)
            pltpu.sync_copy(obuf, out_hbm.at[img])

        for m in range(_IMGS_PER_WORKER):
            run_image(wid * _IMGS_PER_WORKER + m)

    return k(xr)


def kernel(x):
    xr = x.reshape(_N_IMG, _OH, _ROW_ELEMS)
    out = _fold_sc(xr)
    return out.reshape(_B, _C, _H, _W)
